# baseline (speedup 1.0000x reference)
import functools

import jax
import jax.numpy as jnp
from jax import lax
from jax.experimental import pallas as pl
from jax.experimental.pallas import tpu as pltpu
from jax.experimental.pallas import tpu_sc as plsc

NUM_EMB = 100000
DIM = 64
SCALE = 8.0

NC = 2
NS = 16
NW = NC * NS

NBUF = 5


def _transpose_ids_body(i_ref, o_ref):
    x = i_ref[...].T
    o_ref[...] = x.reshape(x.shape[0], 8, 128)


def _transpose_ids(input_ids):
    bsz, tsz = input_ids.shape
    return pl.pallas_call(
        _transpose_ids_body,
        out_shape=jax.ShapeDtypeStruct((tsz, bsz // 128, 128), input_ids.dtype),
        grid=(bsz // 1024,),
        in_specs=[pl.BlockSpec((1024, tsz), lambda j: (j, 0))],
        out_specs=pl.BlockSpec((tsz, 8, 128), lambda j: (0, j, 0)),
    )(input_ids)


def _make_gather(bsz, tsz):
    total = bsz * tsz
    bpw = bsz // NW
    steps = tsz
    assert steps % NBUF == 0
    mesh = plsc.VectorSubcoreMesh(
        core_axis_name="c", subcore_axis_name="s", num_cores=NC, num_subcores=NS
    )

    @functools.partial(
        pl.kernel,
        out_type=jax.ShapeDtypeStruct((total // 2, 2 * DIM), jnp.float32),
        mesh=mesh,
        scratch_types=[
            pltpu.VMEM((tsz, bpw), jnp.int32),
            pltpu.VMEM((NBUF, bpw, DIM), jnp.float32),
            pltpu.SemaphoreType.DMA((NBUF,)),
        ],
        compiler_params=pltpu.CompilerParams(use_tc_tiling_on_sc=False),
    )
    def gather(table_hbm, idsT_hbm, out_hbm, idsT_v, rows_v, sems):
        wid = lax.axis_index("s") * NC + lax.axis_index("c")
        b0 = wid * bpw

        pltpu.sync_copy(idsT_hbm.at[:, wid], idsT_v)

        def idx_ref(i):
            return idsT_v.at[i]

        def fire(i, slot):
            pltpu.async_copy(
                table_hbm.at[idx_ref(i)], rows_v.at[slot], sems.at[slot]
            )

        def drain(i, slot):
            pltpu.make_async_copy(
                table_hbm.at[idx_ref(i)], rows_v.at[slot], sems.at[slot]
            ).wait()
            pltpu.sync_copy(
                rows_v.at[slot],
                out_hbm.at[
                    pl.ds((i // 2) * bsz + b0, bpw), pl.ds((i % 2) * DIM, DIM)
                ],
            )

        for b in range(NBUF):
            fire(b, b)

        @pl.loop(0, steps, step=NBUF)
        def _(g):
            for b in range(NBUF):
                drain(g + b, b)

                @pl.when(g + b + NBUF < steps)
                def _():
                    fire(g + b + NBUF, b)

    return gather


def _finish_body(g_ref, o_ref):
    x = g_ref[...] * SCALE
    eye = jnp.float32(1.0) * (
        lax.broadcasted_iota(jnp.int32, (DIM, DIM), 0)
        == lax.broadcasted_iota(jnp.int32, (DIM, DIM), 1)
    )
    dn = (((1,), (1,)), ((), ()))
    o_ref[0] = lax.dot_general(
        eye, x[:, :DIM], dn,
        precision=lax.Precision.HIGHEST,
        preferred_element_type=jnp.float32,
    )
    o_ref[1] = lax.dot_general(
        eye, x[:, DIM:], dn,
        precision=lax.Precision.HIGHEST,
        preferred_element_type=jnp.float32,
    )


def _finish(g128, bsz, tsz):
    return pl.pallas_call(
        _finish_body,
        out_shape=jax.ShapeDtypeStruct((tsz, DIM, bsz), jnp.float32),
        grid=(tsz // 2,),
        in_specs=[pl.BlockSpec((bsz, 128), lambda i: (i, 0))],
        out_specs=pl.BlockSpec((2, DIM, bsz), lambda i: (i, 0, 0)),
    )(g128)


def kernel(input_ids, weight):
    bsz, tsz = input_ids.shape
    total = bsz * tsz
    ids_t = _transpose_ids(input_ids)
    g128 = _make_gather(bsz, tsz)(weight, ids_t)
    out_phys = _finish(g128, bsz, tsz)
    return jnp.transpose(out_phys, (2, 0, 1))


# speedup vs baseline: 1.7613x; 1.7613x over previous
import functools

import jax
import jax.numpy as jnp
from jax import lax
from jax.experimental import pallas as pl
from jax.experimental.pallas import tpu as pltpu
from jax.experimental.pallas import tpu_sc as plsc

NUM_EMB = 100000
DIM = 64
SCALE = 8.0

NC = 2
NS = 16
NW = NC * NS

NBUF = 5


def _transpose_ids_body(i_ref, o_ref):
    x = i_ref[...].T
    o_ref[...] = x.reshape(x.shape[0], 8, 128)


def _transpose_ids(input_ids):
    bsz, tsz = input_ids.shape
    return pl.pallas_call(
        _transpose_ids_body,
        out_shape=jax.ShapeDtypeStruct((tsz, bsz // 128, 128), input_ids.dtype),
        grid=(bsz // 1024,),
        in_specs=[pl.BlockSpec((1024, tsz), lambda j: (j, 0))],
        out_specs=pl.BlockSpec((tsz, 8, 128), lambda j: (0, j, 0)),
    )(input_ids)


def _make_gather(bsz, tsz):
    total = bsz * tsz
    bpw = bsz // NW
    steps = tsz
    assert steps % NBUF == 0
    mesh = plsc.VectorSubcoreMesh(
        core_axis_name="c", subcore_axis_name="s", num_cores=NC, num_subcores=NS
    )

    @functools.partial(
        pl.kernel,
        out_type=jax.ShapeDtypeStruct((total // 2, 2 * DIM), jnp.float32),
        mesh=mesh,
        scratch_types=[
            pltpu.VMEM((tsz, bpw), jnp.int32),
            pltpu.VMEM((NBUF, bpw, DIM), jnp.float32),
            pltpu.SemaphoreType.DMA((NBUF,)),
        ],
        compiler_params=pltpu.CompilerParams(use_tc_tiling_on_sc=False),
    )
    def gather(table_hbm, idsT_hbm, out_hbm, idsT_v, rows_v, sems):
        wid = lax.axis_index("s") * NC + lax.axis_index("c")
        b0 = wid * bpw

        pltpu.sync_copy(idsT_hbm.at[:, wid], idsT_v)

        def idx_ref(i):
            return idsT_v.at[i]

        def fire(i, slot):
            pltpu.async_copy(
                table_hbm.at[idx_ref(i)], rows_v.at[slot], sems.at[slot]
            )

        def drain(i, slot):
            pltpu.make_async_copy(
                table_hbm.at[idx_ref(i)], rows_v.at[slot], sems.at[slot]
            ).wait()
            pltpu.sync_copy(
                rows_v.at[slot],
                out_hbm.at[
                    pl.ds((i // 2) * bsz + b0, bpw), pl.ds((i % 2) * DIM, DIM)
                ],
            )

        for b in range(NBUF):
            fire(b, b)

        @pl.loop(0, steps, step=NBUF)
        def _(g):
            for b in range(NBUF):
                drain(g + b, b)

                @pl.when(g + b + NBUF < steps)
                def _():
                    fire(g + b + NBUF, b)

    return gather


def _finish_body(g_ref, o_ref):
    y = (g_ref[...] * SCALE).T
    o_ref[0] = y[:DIM]
    o_ref[1] = y[DIM:]


def _finish(g128, bsz, tsz):
    return pl.pallas_call(
        _finish_body,
        out_shape=jax.ShapeDtypeStruct((tsz, DIM, bsz), jnp.float32),
        grid=(tsz // 2,),
        in_specs=[pl.BlockSpec((bsz, 128), lambda i: (i, 0))],
        out_specs=pl.BlockSpec((2, DIM, bsz), lambda i: (i, 0, 0)),
    )(g128)


def kernel(input_ids, weight):
    bsz, tsz = input_ids.shape
    total = bsz * tsz
    ids_t = _transpose_ids(input_ids)
    g128 = _make_gather(bsz, tsz)(weight, ids_t)
    out_phys = _finish(g128, bsz, tsz)
    return jnp.transpose(out_phys, (2, 0, 1))
